# baseline (speedup 1.0000x reference)
import functools

import jax
import jax.numpy as jnp
from jax import lax
from jax.experimental import pallas as pl
from jax.experimental.pallas import tpu as pltpu

_LOG2 = 0.6931471805599453
_LOG2E = 1.4426950408889634
_NI = 3
_GAMMA = 10.0
_MB = 16
_T = 8
_RS = (_GAMMA * _LOG2E) ** 0.5


def _ssp(x):
    return jnp.maximum(x, 0.0) + jnp.log1p(jnp.exp(-jnp.abs(x))) - _LOG2


def _schnet_kernel(z_ref, r_ref, sw_ref, emb0_ref, emb1_ref, wf1_ref,
                   bf1_ref, wf2_ref, bf2_ref, win_ref, bin_ref, wo1_ref,
                   bo1_ref, wo2_ref, bo2_ref, wa1_ref, ba1_ref, wa2_ref,
                   out_ref, *, A, G, NF, F, NC):
    f32 = jnp.float32
    W = 2 * G
    T = _T
    TA = T * A

    zrow = z_ref[0]

    it2 = lax.broadcasted_iota(jnp.int32, (NC, TA), 0)
    oh0 = (it2 == zrow[:, :TA]).astype(f32)
    oh1 = (it2 == zrow[:, TA:]).astype(f32)
    x = (lax.dot_general(oh0, emb0_ref[...], (((0,), (0,)), ((), ())),
                         preferred_element_type=f32)
         + lax.dot_general(oh1, emb1_ref[...], (((0,), (0,)), ((), ())),
                           preferred_element_type=f32))

    rb = r_ref[...] * _RS
    rp = jnp.concatenate(
        [jnp.concatenate(
            [jnp.broadcast_to(rb[t * A:(t + 1) * A][:, :, None],
                              (A, A, G)),
             jnp.broadcast_to(rb[(T + t) * A:(T + t + 1) * A][:, :, None],
                              (A, A, G))],
            axis=2).reshape(A * A, W)
         for t in range(T)],
        axis=0)
    centers = ((lax.broadcasted_iota(jnp.int32, (1, W), 1) & (G - 1))
               .astype(f32) * (_RS / (G - 1)))
    rbf = jnp.exp2((rp - centers) * (centers - rp))

    a1 = jnp.dot(rbf, wf1_ref[...], preferred_element_type=f32) + bf1_ref[...]
    h = jnp.log(1.0 + jnp.exp(a1)) - _LOG2
    a2 = jnp.dot(h, wf2_ref[...], preferred_element_type=f32) + bf2_ref[...]
    wf = jnp.log(1.0 + jnp.exp(a2)) - _LOG2
    wf4 = wf.reshape(T, A, A, W)

    for _ in range(_NI):
        xf = jnp.dot(x, win_ref[...], preferred_element_type=f32) + bin_ref[...]
        xf4 = xf.reshape(T, 1, A, W)
        p = (wf4 * xf4).reshape(T * A * A, W)
        y = jnp.concatenate(
            [jnp.dot(sw_ref[...], p[t * A * A:(t + 1) * A * A],
                     preferred_element_type=f32) for t in range(T)],
            axis=0)
        v = _ssp(jnp.dot(y, wo1_ref[...], preferred_element_type=f32)
                 + bo1_ref[...])
        v = jnp.dot(v, wo2_ref[...], preferred_element_type=f32) + bo2_ref[...]
        x = x + v

    xa = _ssp(jnp.dot(x, wa1_ref[...], preferred_element_type=f32)
              + ba1_ref[...])
    xt = jnp.concatenate([xa[:, :F], xa[:, F:]], axis=0)
    o = lax.dot_general(wa2_ref[...], xt,
                        (((1,), (1,)), ((), ())),
                        preferred_element_type=f32)
    out_ref[0] = o


def kernel(z, r, emb, W_f1, b_f1, W_f2, b_f2, W_in, b_in, W_o1, b_o1,
           W_o2, b_o2, W_a1, b_a1, W_a2, b_a2):
    B, A = z.shape
    G, NF = W_f1.shape
    F = emb.shape[1]
    NC = 128
    MB = _MB
    M = MB * A
    f32 = jnp.float32

    z3 = z.astype(jnp.int32).reshape(B // MB, 1, M)
    r2 = r.reshape(B * A, A)
    emb_pad = jnp.zeros((NC, F), f32).at[:emb.shape[0]].set(emb)
    zf = jnp.zeros((NC, F), f32)
    emb0 = jnp.concatenate([emb_pad, zf], axis=1)
    emb1 = jnp.concatenate([zf, emb_pad], axis=1)

    def bd(w):
        n, m = w.shape
        out = jnp.zeros((2 * n, 2 * m), f32)
        return out.at[:n, :m].set(w).at[n:, m:].set(w)

    wrow = lambda b: jnp.tile(b.reshape(1, -1).astype(f32), (1, 2))

    Sw = jnp.repeat(jnp.eye(A, dtype=f32), A, axis=1)

    full = lambda shape: pl.BlockSpec(shape, lambda b: (0,) * len(shape))

    out = pl.pallas_call(
        functools.partial(_schnet_kernel, A=A, G=G, NF=NF, F=F, NC=NC),
        grid=(B // MB,),
        in_specs=[
            pl.BlockSpec((1, 1, M), lambda b: (b, 0, 0)),
            pl.BlockSpec((M, A), lambda b: (b, 0)),
            full((A, A * A)),
            full((NC, 2 * F)), full((NC, 2 * F)),
            full((2 * G, 2 * NF)), full((1, 2 * NF)),
            full((2 * NF, 2 * NF)), full((1, 2 * NF)),
            full((2 * F, 2 * NF)), full((1, 2 * NF)),
            full((2 * NF, 2 * F)), full((1, 2 * F)),
            full((2 * F, 2 * F)), full((1, 2 * F)),
            full((2 * F, 2 * F)), full((1, 2 * F)),
            full((1, F)),
        ],
        out_specs=pl.BlockSpec((1, 1, M), lambda b: (b, 0, 0)),
        out_shape=jax.ShapeDtypeStruct((B // MB, 1, M), f32),
        compiler_params=pltpu.CompilerParams(
            dimension_semantics=("parallel",)),
    )(z3, r2, Sw, emb0, emb1, bd(W_f1), wrow(b_f1), bd(W_f2), wrow(b_f2),
      bd(W_in), wrow(b_in), bd(W_o1), wrow(b_o1), bd(W_o2), wrow(b_o2),
      bd(W_a1), wrow(b_a1), W_a2.reshape(1, F))

    return out.reshape(B, A, 1) + b_a2[0]


# speedup vs baseline: 1.0337x; 1.0337x over previous
import functools

import jax
import jax.numpy as jnp
from jax import lax
from jax.experimental import pallas as pl
from jax.experimental.pallas import tpu as pltpu

_LOG2 = 0.6931471805599453
_LOG2E = 1.4426950408889634
_NI = 3
_GAMMA = 10.0
_MB = 16
_T = 8
_RS = (_GAMMA * _LOG2E) ** 0.5


def _ssp(x):
    return jnp.maximum(x, 0.0) + jnp.log1p(jnp.exp(-jnp.abs(x))) - _LOG2


def _schnet_kernel(z_ref, r_ref, sw_ref, emb0_ref, emb1_ref, wf1_ref,
                   bf1_ref, wf2_ref, bf2_ref, win_ref, bin_ref, wo1_ref,
                   bo1_ref, wo2_ref, bo2_ref, wa1_ref, ba1_ref, wa2_ref,
                   out_ref, *, A, G, NF, F, NC):
    f32 = jnp.float32
    W = 2 * G
    T = _T
    TA = T * A

    zrow = z_ref[0]

    it2 = lax.broadcasted_iota(jnp.int32, (NC, TA), 0)
    oh0 = (it2 == zrow[:, :TA]).astype(f32)
    oh1 = (it2 == zrow[:, TA:]).astype(f32)
    x = (lax.dot_general(oh0, emb0_ref[...], (((0,), (0,)), ((), ())),
                         preferred_element_type=f32)
         + lax.dot_general(oh1, emb1_ref[...], (((0,), (0,)), ((), ())),
                           preferred_element_type=f32))

    rb = r_ref[...]
    rp = jnp.concatenate(
        [jnp.concatenate(
            [jnp.broadcast_to(rb[t * A:(t + 1) * A][:, :, None],
                              (A, A, G)),
             jnp.broadcast_to(rb[(T + t) * A:(T + t + 1) * A][:, :, None],
                              (A, A, G))],
            axis=2).reshape(A * A, W)
         for t in range(T)],
        axis=0)
    centers = ((lax.broadcasted_iota(jnp.int32, (1, W), 1) & (G - 1))
               .astype(f32) * (_RS / (G - 1)))
    rbf = jnp.exp2((rp - centers) * (centers - rp))

    a1 = jnp.dot(rbf, wf1_ref[...], preferred_element_type=f32) + bf1_ref[...]
    h = jnp.log(1.0 + jnp.exp(a1)) - _LOG2
    a2 = jnp.dot(h, wf2_ref[...], preferred_element_type=f32) + bf2_ref[...]
    wf = jnp.log(1.0 + jnp.exp(a2)) - _LOG2
    wf4 = wf.reshape(T, A, A, W)

    for _ in range(_NI):
        xf = jnp.dot(x, win_ref[...], preferred_element_type=f32) + bin_ref[...]
        xf4 = xf.reshape(T, 1, A, W)
        p = (wf4 * xf4).reshape(T * A * A, W)
        y = jnp.concatenate(
            [jnp.dot(sw_ref[...], p[t * A * A:(t + 1) * A * A],
                     preferred_element_type=f32) for t in range(T)],
            axis=0)
        v = _ssp(jnp.dot(y, wo1_ref[...], preferred_element_type=f32)
                 + bo1_ref[...])
        v = jnp.dot(v, wo2_ref[...], preferred_element_type=f32) + bo2_ref[...]
        x = x + v

    xa = _ssp(jnp.dot(x, wa1_ref[...], preferred_element_type=f32)
              + ba1_ref[...])
    xt = jnp.concatenate([xa[:, :F], xa[:, F:]], axis=0)
    o = lax.dot_general(wa2_ref[...], xt,
                        (((1,), (1,)), ((), ())),
                        preferred_element_type=f32)
    out_ref[0] = o


def kernel(z, r, emb, W_f1, b_f1, W_f2, b_f2, W_in, b_in, W_o1, b_o1,
           W_o2, b_o2, W_a1, b_a1, W_a2, b_a2):
    B, A = z.shape
    G, NF = W_f1.shape
    F = emb.shape[1]
    NC = 128
    MB = _MB
    M = MB * A
    f32 = jnp.float32

    z3 = z.astype(jnp.int32).reshape(B // MB, 1, M)
    r2 = (r * _RS).reshape(B * A, A)
    emb_pad = jnp.zeros((NC, F), f32).at[:emb.shape[0]].set(emb)
    zf = jnp.zeros((NC, F), f32)
    emb0 = jnp.concatenate([emb_pad, zf], axis=1)
    emb1 = jnp.concatenate([zf, emb_pad], axis=1)

    def bd(w):
        n, m = w.shape
        out = jnp.zeros((2 * n, 2 * m), f32)
        return out.at[:n, :m].set(w).at[n:, m:].set(w)

    wrow = lambda b: jnp.tile(b.reshape(1, -1).astype(f32), (1, 2))

    Sw = jnp.repeat(jnp.eye(A, dtype=f32), A, axis=1)

    full = lambda shape: pl.BlockSpec(shape, lambda b: (0,) * len(shape))

    out = pl.pallas_call(
        functools.partial(_schnet_kernel, A=A, G=G, NF=NF, F=F, NC=NC),
        grid=(B // MB,),
        in_specs=[
            pl.BlockSpec((1, 1, M), lambda b: (b, 0, 0)),
            pl.BlockSpec((M, A), lambda b: (b, 0)),
            full((A, A * A)),
            full((NC, 2 * F)), full((NC, 2 * F)),
            full((2 * G, 2 * NF)), full((1, 2 * NF)),
            full((2 * NF, 2 * NF)), full((1, 2 * NF)),
            full((2 * F, 2 * NF)), full((1, 2 * NF)),
            full((2 * NF, 2 * F)), full((1, 2 * F)),
            full((2 * F, 2 * F)), full((1, 2 * F)),
            full((2 * F, 2 * F)), full((1, 2 * F)),
            full((1, F)),
        ],
        out_specs=pl.BlockSpec((1, 1, M), lambda b: (b, 0, 0)),
        out_shape=jax.ShapeDtypeStruct((B // MB, 1, M), f32),
        compiler_params=pltpu.CompilerParams(
            dimension_semantics=("parallel",)),
    )(z3, r2, Sw, emb0, emb1, bd(W_f1), wrow(b_f1), bd(W_f2), wrow(b_f2),
      bd(W_in), wrow(b_in), bd(W_o1), wrow(b_o1), bd(W_o2), wrow(b_o2),
      bd(W_a1), wrow(b_a1), W_a2.reshape(1, F))

    return out.reshape(B, A, 1) + b_a2[0]
